# baseline (speedup 1.0000x reference)
import jax
import jax.numpy as jnp
from jax.experimental import pallas as pl
from jax.experimental.pallas import tpu as pltpu
from jax.experimental.pallas import tpu_sc as plsc

_CHUNK = 14848
_NSTATIC = 6
_TAIL = 11008


def kernel(indexes, weight):
    vocab = weight.shape[0]
    batch = indexes.shape[0]
    idx = indexes.astype(jnp.int32)
    mesh = plsc.VectorSubcoreMesh(core_axis_name="core",
                                  subcore_axis_name="subcore")
    n_workers = 32
    tpw = (batch // 8) // n_workers

    @pl.kernel(out_type=jax.ShapeDtypeStruct((batch, vocab), jnp.float32),
               mesh=mesh,
               scratch_types=[
                   pltpu.VMEM((8, _CHUNK), jnp.float32),
                   pltpu.VMEM((8, 8, 128), jnp.float32),
                   pltpu.VMEM((batch,), jnp.int32),
                   pltpu.SemaphoreType.DMA,
                   pltpu.SemaphoreType.DMA,
               ] + [pltpu.SemaphoreType.DMA] * tpw)
    def sc_kernel(i_hbm, o_hbm, zb, hb, idxv, sem_p, sem_i, *sem_zs):
        core = jax.lax.axis_index("core")
        sub = jax.lax.axis_index("subcore")
        wid = core * 16 + sub
        zoff = wid * 0

        pltpu.async_copy(i_hbm, idxv, sem_i).wait()

        zeros16 = jnp.zeros((16,), jnp.float32)
        iota = jax.lax.iota(jnp.int32, 16)

        for r in range(8):
            @pl.loop(0, _CHUNK, step=16)
            def _(k, r=r):
                zb[r, pl.ds(k, 16)] = zeros16

        rt8s = [pl.multiple_of((wid * tpw + t) * 8, 8) for t in range(tpw)]

        zdescs = [[] for _ in range(tpw)]
        for t in range(tpw):
            for k in range(_NSTATIC):
                d = pltpu.make_async_copy(
                    zb,
                    o_hbm.at[pl.ds(rt8s[t], 8), pl.ds(k * _CHUNK, _CHUNK)],
                    sem_zs[t])
                d.start()
                zdescs[t].append(d)
            col = pl.multiple_of(zoff + _NSTATIC * _CHUNK, 128)
            d = pltpu.make_async_copy(
                zb.at[:, pl.ds(0, _TAIL)],
                o_hbm.at[pl.ds(rt8s[t], 8), pl.ds(col, _TAIL)],
                sem_zs[t])
            d.start()
            zdescs[t].append(d)

        pdescs = []
        for t in range(tpw):
            rt8 = rt8s[t]

            for d in pdescs:
                d.wait()
            pdescs = []

            for i in range(8):
                for j in range(8):
                    @pl.loop(0, 128, step=16)
                    def _(k, i=i, j=j):
                        hb[i, j, pl.ds(k, 16)] = zeros16

            cs = [idxv[pl.ds(rt8 + j, 1)][0] for j in range(8)]
            bs = [(c // 128) * 128 for c in cs]
            for i in range(8):
                for j in range(8):
                    lane = cs[j] - bs[j]
                    sj = (lane // 16) * 16
                    tgt = jnp.where(bs[i] == bs[j], lane - sj, -1)
                    vec = jnp.where(iota == tgt, 1.0, 0.0).astype(jnp.float32)
                    hb[i, j, pl.ds(sj, 16)] = vec

            for d in zdescs[t]:
                d.wait()

            for i in range(8):
                coli = pl.multiple_of(bs[i], 128)
                d = pltpu.make_async_copy(
                    hb.at[i],
                    o_hbm.at[pl.ds(rt8, 8), pl.ds(coli, 128)],
                    sem_p)
                d.start()
                pdescs.append(d)

        for d in pdescs:
            d.wait()

    return sc_kernel(idx)


# speedup vs baseline: 1.0433x; 1.0433x over previous
import jax
import jax.numpy as jnp
from jax.experimental import pallas as pl
from jax.experimental.pallas import tpu as pltpu
from jax.experimental.pallas import tpu_sc as plsc

_CHUNK = 2048
_NSTATIC = 48
_TAIL = 1792


def kernel(indexes, weight):
    vocab = weight.shape[0]
    batch = indexes.shape[0]
    idx = indexes.astype(jnp.int32)
    mesh = plsc.VectorSubcoreMesh(core_axis_name="core",
                                  subcore_axis_name="subcore")
    n_workers = 32
    tpw = (batch // 8) // n_workers

    @pl.kernel(out_type=jax.ShapeDtypeStruct((batch, vocab), jnp.float32),
               mesh=mesh,
               scratch_types=[
                   pltpu.VMEM((8, _CHUNK), jnp.float32),
                   pltpu.VMEM((8, 8, 128), jnp.float32),
                   pltpu.VMEM((batch,), jnp.int32),
                   pltpu.SemaphoreType.DMA,
                   pltpu.SemaphoreType.DMA,
               ] + [pltpu.SemaphoreType.DMA] * tpw)
    def sc_kernel(i_hbm, o_hbm, zb, hb, idxv, sem_p, sem_i, *sem_zs):
        core = jax.lax.axis_index("core")
        sub = jax.lax.axis_index("subcore")
        wid = core * 16 + sub
        zoff = wid * 0

        pltpu.async_copy(i_hbm, idxv, sem_i).wait()

        zeros16 = jnp.zeros((16,), jnp.float32)
        iota = jax.lax.iota(jnp.int32, 16)

        for r in range(8):
            @pl.loop(0, _CHUNK, step=16)
            def _(k, r=r):
                zb[r, pl.ds(k, 16)] = zeros16

        rt8s = [pl.multiple_of((wid * tpw + t) * 8, 8) for t in range(tpw)]

        zdescs = [[] for _ in range(tpw)]
        for t in range(tpw):
            for k in range(_NSTATIC):
                d = pltpu.make_async_copy(
                    zb,
                    o_hbm.at[pl.ds(rt8s[t], 8), pl.ds(k * _CHUNK, _CHUNK)],
                    sem_zs[t])
                d.start()
                zdescs[t].append(d)
            col = pl.multiple_of(zoff + _NSTATIC * _CHUNK, 128)
            d = pltpu.make_async_copy(
                zb.at[:, pl.ds(0, _TAIL)],
                o_hbm.at[pl.ds(rt8s[t], 8), pl.ds(col, _TAIL)],
                sem_zs[t])
            d.start()
            zdescs[t].append(d)

        pdescs = []
        for t in range(tpw):
            rt8 = rt8s[t]

            for d in pdescs:
                d.wait()
            pdescs = []

            for i in range(8):
                for j in range(8):
                    @pl.loop(0, 128, step=16)
                    def _(k, i=i, j=j):
                        hb[i, j, pl.ds(k, 16)] = zeros16

            cs = [idxv[pl.ds(rt8 + j, 1)][0] for j in range(8)]
            bs = [(c // 128) * 128 for c in cs]
            for i in range(8):
                for j in range(8):
                    lane = cs[j] - bs[j]
                    sj = (lane // 16) * 16
                    tgt = jnp.where(bs[i] == bs[j], lane - sj, -1)
                    vec = jnp.where(iota == tgt, 1.0, 0.0).astype(jnp.float32)
                    hb[i, j, pl.ds(sj, 16)] = vec

            for d in zdescs[t]:
                d.wait()

            for i in range(8):
                coli = pl.multiple_of(bs[i], 128)
                d = pltpu.make_async_copy(
                    hb.at[i],
                    o_hbm.at[pl.ds(rt8, 8), pl.ds(coli, 128)],
                    sem_p)
                d.start()
                pdescs.append(d)

        for d in pdescs:
            d.wait()

    return sc_kernel(idx)
